# Initial kernel scaffold; baseline (speedup 1.0000x reference)
import jax
import jax.numpy as jnp
from jax.experimental import pallas as pl


def kernel(h, edge_index, e, graph_ids, W_emb, b_emb, W_self, b_self, W_neigh, b_neigh, gamma, beta, p_pos, p_neg, W_fc):
    raise NotImplementedError("write your pallas kernel here")


# speedup vs baseline: 5.4293x; 5.4293x over previous
import jax
import jax.numpy as jnp
from jax import lax
from jax.experimental import pallas as pl
from jax.experimental.pallas import tpu as pltpu
from jax.experimental.pallas import tpu_sc as plsc

_N = 10000
_E = 320000
_H = 128
_B = 64
_P = 5

_NC = 2
_NS = 16
_NW = _NC * _NS

_CH = 128
_NCHUNK = _E // _CH
_BASE_CHUNKS = _NCHUNK // _NW
_EXTRA = _NCHUNK - _BASE_CHUNKS * _NW

_NP = 10240
_RPS = _NP // _NS
_ZR = 128


def _zero_fill(buf, rows, cols, val):
    def zrow(i, _):
        def zcol(j, _):
            buf[i, pl.ds(j * 16, 16)] = jnp.full((16,), val, jnp.float32)
            return 0
        return lax.fori_loop(0, cols // 16, zcol, 0)
    lax.fori_loop(0, rows, zrow, 0)


def _sc_mesh():
    return plsc.VectorSubcoreMesh(
        core_axis_name="c", subcore_axis_name="s",
        num_cores=_NC, num_subcores=_NS)


def _seg_rows_body(x_hbm, src_hbm, dst_hbm, out_hbm,
                   idx_s, idx_d, rows, zbuf, agg, sem):
    c = lax.axis_index("c")
    s = lax.axis_index("s")
    wid = s * _NC + c

    _zero_fill(zbuf, _ZR, _H, 0.0)
    def zcopy(i, _):
        pltpu.sync_copy(zbuf, agg.at[pl.ds(s * _RPS + i * _ZR, _ZR)])
        return 0
    lax.fori_loop(0, _RPS // _ZR, zcopy, 0)
    plsc.subcore_barrier()

    nb = _BASE_CHUNKS + jnp.where(wid < _EXTRA, 1, 0)

    def step(j, _):
        q = wid + j * _NW
        pltpu.sync_copy(src_hbm.at[q], idx_s)
        pltpu.sync_copy(dst_hbm.at[q], idx_d.at[0])
        pltpu.async_copy(x_hbm.at[idx_s], rows, sem).wait()
        pltpu.sync_copy(rows, agg.at[idx_d.at[0]], add=True)
        return 0
    lax.fori_loop(0, nb, step, 0)

    plsc.subcore_barrier()
    pltpu.sync_copy(agg.at[pl.ds(s * _RPS, _RPS)],
                    out_hbm.at[pl.ds(c * _NP + s * _RPS, _RPS)])


def _sc_segment_rows(x, src2d, dst2d):
    return pl.kernel(
        _seg_rows_body,
        out_type=jax.ShapeDtypeStruct((_NC * _NP, _H), jnp.float32),
        mesh=_sc_mesh(),
        scratch_types=[
            pltpu.VMEM((_CH,), jnp.int32),
            pltpu.VMEM((1, _CH), jnp.int32),
            pltpu.VMEM((_CH, _H), jnp.float32),
            pltpu.VMEM((_ZR, _H), jnp.float32),
            pltpu.VMEM_SHARED((_NP, _H), jnp.float32),
            pltpu.SemaphoreType.DMA,
        ],
    )(x, src2d, dst2d)


def _deg_body(dst_hbm, out_hbm, idx_d, ones, zbuf, agg):
    c = lax.axis_index("c")
    s = lax.axis_index("s")
    wid = s * _NC + c

    _zero_fill(zbuf, _ZR, _H, 0.0)
    _zero_fill(ones, _CH, _H, 1.0)
    def zcopy(i, _):
        pltpu.sync_copy(zbuf, agg.at[pl.ds(s * _RPS + i * _ZR, _ZR)])
        return 0
    lax.fori_loop(0, _RPS // _ZR, zcopy, 0)
    plsc.subcore_barrier()

    nb = _BASE_CHUNKS + jnp.where(wid < _EXTRA, 1, 0)

    def step(j, _):
        q = wid + j * _NW
        pltpu.sync_copy(dst_hbm.at[q], idx_d.at[0])
        pltpu.sync_copy(ones, agg.at[idx_d.at[0]], add=True)
        return 0
    lax.fori_loop(0, nb, step, 0)

    plsc.subcore_barrier()
    pltpu.sync_copy(agg.at[pl.ds(s * _RPS, _RPS)],
                    out_hbm.at[pl.ds(c * _NP + s * _RPS, _RPS)])


def _sc_degrees(dst2d):
    return pl.kernel(
        _deg_body,
        out_type=jax.ShapeDtypeStruct((_NC * _NP, _H), jnp.float32),
        mesh=_sc_mesh(),
        scratch_types=[
            pltpu.VMEM((1, _CH), jnp.int32),
            pltpu.VMEM((_CH, _H), jnp.float32),
            pltpu.VMEM((_ZR, _H), jnp.float32),
            pltpu.VMEM_SHARED((_NP, _H), jnp.float32),
        ],
    )(dst2d)


def _embed_body(h_ref, w_ref, b_ref, o_ref):
    o_ref[...] = lax.dot_general(
        h_ref[...], w_ref[...], (((1,), (0,)), ((), ())),
        preferred_element_type=jnp.float32) + b_ref[...]


def _tc_embed(h, W_emb, b_emb2d):
    return pl.pallas_call(
        _embed_body,
        out_shape=jax.ShapeDtypeStruct((_N, _H), jnp.float32),
    )(h, W_emb, b_emb2d)


def _layer_body(x_ref, parts_ref, degp_ref, ws_ref, bs_ref, wn_ref, bn_ref,
                g_ref, bt_ref, o_ref):
    x = x_ref[...]
    deg = degp_ref[0:_N, 0:1] + degp_ref[_NP:_NP + _N, 0:1]
    rdeg = 1.0 / jnp.maximum(deg, 1.0)
    agg = (parts_ref[0:_N, :] + parts_ref[_NP:_NP + _N, :]) * rdeg
    out = (lax.dot_general(x, ws_ref[...], (((1,), (0,)), ((), ())),
                           preferred_element_type=jnp.float32)
           + bs_ref[...]
           + lax.dot_general(agg, wn_ref[...], (((1,), (0,)), ((), ())),
                             preferred_element_type=jnp.float32)
           + bn_ref[...])
    out = jnp.maximum(out, 0.0)
    mu = jnp.mean(out, axis=0, keepdims=True)
    var = jnp.mean((out - mu) ** 2, axis=0, keepdims=True)
    out = g_ref[...] * (out - mu) / jnp.sqrt(var + 1e-5) + bt_ref[...]
    o_ref[...] = x + out


def _tc_layer(x, parts, degp, Ws, bs2d, Wn, bn2d, g2d, bt2d):
    return pl.pallas_call(
        _layer_body,
        out_shape=jax.ShapeDtypeStruct((_N, _H), jnp.float32),
    )(x, parts, degp, Ws, bs2d, Wn, bn2d, g2d, bt2d)


def _head_body(x_ref, gid_ref, pp_ref, pn_ref, wfc_ref, o_ref):
    x = x_ref[...]
    ids = gid_ref[...]
    iota = lax.broadcasted_iota(jnp.int32, (_N, _B), 1)
    mask = (ids == iota).astype(jnp.float32)
    cnt = jnp.sum(mask, axis=0, keepdims=True)
    hgs = lax.dot_general(mask, x, (((0,), (0,)), ((), ())),
                          preferred_element_type=jnp.float32)
    hg = hgs / jnp.maximum(cnt, 1.0).reshape(_B, 1)
    cols = []
    for i in range(_P):
        dp = jnp.sum((hg - pp_ref[i:i + 1, :]) ** 2, axis=1, keepdims=True)
        cols.append(dp)
    for i in range(_P):
        dn = jnp.sum((hg - pn_ref[i:i + 1, :]) ** 2, axis=1, keepdims=True)
        cols.append(dn)
    d = jnp.concatenate(cols, axis=1)
    ss = jnp.log((d + 1.0) / (d + 1e-12))
    y = lax.dot_general(ss, wfc_ref[...], (((1,), (1,)), ((), ())),
                        preferred_element_type=jnp.float32)
    o_ref[...] = 1.0 / (1.0 + jnp.exp(-y))


def _tc_head(x, gid2d, p_pos, p_neg, W_fc):
    return pl.pallas_call(
        _head_body,
        out_shape=jax.ShapeDtypeStruct((_B, 2), jnp.float32),
    )(x, gid2d, p_pos, p_neg, W_fc)


@jax.jit
def kernel(h, edge_index, e, graph_ids, W_emb, b_emb, W_self, b_self,
           W_neigh, b_neigh, gamma, beta, p_pos, p_neg, W_fc):
    src2d = edge_index[0].reshape(_NCHUNK, _CH)
    dst2d = edge_index[1].reshape(_NCHUNK, _CH)
    gid2d = graph_ids.reshape(_N, 1)

    x = _tc_embed(h, W_emb, b_emb.reshape(1, _H))
    degp = _sc_degrees(dst2d)
    for l in range(3):
        parts = _sc_segment_rows(x, src2d, dst2d)
        x = _tc_layer(x, parts, degp,
                      W_self[l], b_self[l].reshape(1, _H),
                      W_neigh[l], b_neigh[l].reshape(1, _H),
                      gamma[l].reshape(1, _H), beta[l].reshape(1, _H))
    return _tc_head(x, gid2d, p_pos, p_neg, W_fc)
